# baseline (speedup 1.0000x reference)
import functools

import jax
import jax.numpy as jnp
from jax import lax
from jax.experimental import pallas as pl
from jax.experimental.pallas import tpu as pltpu
from jax.experimental.pallas import tpu_sc as plsc

E, CIN, COUT, D = 8192, 4, 8, 512
L = 16
NC, NS = 2, 16
NW = NC * NS
EPW = E // NW
C = 4
NCH = EPW // C
NB = 2
JPE = D // L
INV_SQRT3 = 0.5773502691896258


def _body(x_hbm, y_hbm, o_hbm, xy, ov, sg0, sg1, so0, so1):
    sg = (sg0, sg1)
    so = (so0, so1)
    wid = lax.axis_index("s") * NC + lax.axis_index("c")
    base = wid * EPW

    def _fire_gathers(b, off):
        pltpu.async_copy(x_hbm.at[pl.ds(off, C)], xy.at[b, pl.ds(0, C)], sg[b])
        pltpu.async_copy(y_hbm.at[pl.ds(off, C)], xy.at[b, pl.ds(C, C)], sg[b])

    for b in range(NB):
        _fire_gathers(b, base + b * C)

    def round_body(g, carry):
        for b in range(NB):
            ci = g * NB + b
            off = base + ci * C

            pltpu.make_async_copy(
                x_hbm.at[pl.ds(0, 2 * C)], xy.at[b], sg[b]).wait()

            @pl.when(g > 0)
            def _():
                pltpu.make_async_copy(
                    ov.at[b], o_hbm.at[pl.ds(base, C)], so[b]).wait()

            for e in range(C):
                for j in range(JPE):
                    s = pl.ds(j * L, L)
                    x0 = xy[b, e, 0, s]
                    x1 = xy[b, e, 1, s]
                    x2 = xy[b, e, 2, s]
                    x3 = xy[b, e, 3, s]
                    y0 = xy[b, C + e, 0, s]
                    y1 = xy[b, C + e, 1, s]
                    y2 = xy[b, C + e, 2, s]
                    y3 = xy[b, C + e, 3, s]
                    ov[b, e, 0, s] = x0 * y0
                    ov[b, e, 1, s] = x0 * y1
                    ov[b, e, 2, s] = x0 * y2
                    ov[b, e, 3, s] = x0 * y3
                    ov[b, e, 4, s] = x1 * y0
                    ov[b, e, 5, s] = x2 * y0
                    ov[b, e, 6, s] = x3 * y0
                    ov[b, e, 7, s] = (x1 * y1 + x2 * y2 + x3 * y3) * INV_SQRT3

            pltpu.async_copy(ov.at[b], o_hbm.at[pl.ds(off, C)], so[b])

            @pl.when(ci + NB < NCH)
            def _():
                _fire_gathers(b, off + NB * C)

        return carry

    lax.fori_loop(0, NCH // NB, round_body, 0)

    for b in range(NB):
        pltpu.make_async_copy(ov.at[b], o_hbm.at[pl.ds(base, C)], so[b]).wait()


_tp = functools.partial(
    pl.kernel,
    mesh=plsc.VectorSubcoreMesh(core_axis_name="c", subcore_axis_name="s"),
    out_type=jax.ShapeDtypeStruct((E, COUT, D), jnp.float32),
    scratch_types=[
        pltpu.VMEM((NB, 2 * C, CIN, D), jnp.float32),
        pltpu.VMEM((NB, C, COUT, D), jnp.float32),
        pltpu.SemaphoreType.DMA,
        pltpu.SemaphoreType.DMA,
        pltpu.SemaphoreType.DMA,
        pltpu.SemaphoreType.DMA,
    ],
)(_body)


def kernel(x, y):
    return _tp(x, y)


# speedup vs baseline: 1.1460x; 1.1460x over previous
import functools

import jax
import jax.numpy as jnp
from jax import lax
from jax.experimental import pallas as pl
from jax.experimental.pallas import tpu as pltpu
from jax.experimental.pallas import tpu_sc as plsc

E, CIN, COUT, D = 8192, 4, 8, 512
L = 16
NC, NS = 2, 16
NW = NC * NS
EPW = E // NW
C = 4
NCH = EPW // C
NB = 2
JPE = D // L
INV_SQRT3 = 0.5773502691896258


def _body(x_hbm, y_hbm, o_hbm, xy, ov, sg0, sg1, so0, so1):
    sg = (sg0, sg1)
    so = (so0, so1)
    wid = lax.axis_index("s") * NC + lax.axis_index("c")
    base = wid * EPW

    def _fire_gathers(b, off):
        pltpu.async_copy(x_hbm.at[pl.ds(off, C)], xy.at[b, pl.ds(0, C)], sg[b])
        pltpu.async_copy(y_hbm.at[pl.ds(off, C)], xy.at[b, pl.ds(C, C)], sg[b])

    for b in range(NB):
        _fire_gathers(b, base + b * C)

    def round_body(g, carry):
        for b in range(NB):
            ci = g * NB + b
            off = base + ci * C

            pltpu.make_async_copy(
                x_hbm.at[pl.ds(0, 2 * C)], xy.at[b], sg[b]).wait()

            @pl.when(g > 0)
            def _():
                pltpu.make_async_copy(
                    ov.at[b], o_hbm.at[pl.ds(base, C)], so[b]).wait()

            def _edge(e, carry3):
                for j in range(JPE):
                    s = pl.ds(j * L, L)
                    x0 = xy[b, e, 0, s]
                    x1 = xy[b, e, 1, s]
                    x2 = xy[b, e, 2, s]
                    x3 = xy[b, e, 3, s]
                    y0 = xy[b, C + e, 0, s]
                    y1 = xy[b, C + e, 1, s]
                    y2 = xy[b, C + e, 2, s]
                    y3 = xy[b, C + e, 3, s]
                    ov[b, e, 0, s] = x0 * y0
                    ov[b, e, 1, s] = x0 * y1
                    ov[b, e, 2, s] = x0 * y2
                    ov[b, e, 3, s] = x0 * y3
                    ov[b, e, 4, s] = x1 * y0
                    ov[b, e, 5, s] = x2 * y0
                    ov[b, e, 6, s] = x3 * y0
                    ov[b, e, 7, s] = (x1 * y1 + x2 * y2 + x3 * y3) * INV_SQRT3
                return carry3

            lax.fori_loop(0, C, _edge, 0)

            pltpu.async_copy(ov.at[b], o_hbm.at[pl.ds(off, C)], so[b])

            @pl.when(ci + NB < NCH)
            def _():
                _fire_gathers(b, off + NB * C)

        return carry

    lax.fori_loop(0, NCH // NB, round_body, 0)

    for b in range(NB):
        pltpu.make_async_copy(ov.at[b], o_hbm.at[pl.ds(base, C)], so[b]).wait()


_tp = functools.partial(
    pl.kernel,
    mesh=plsc.VectorSubcoreMesh(core_axis_name="c", subcore_axis_name="s"),
    out_type=jax.ShapeDtypeStruct((E, COUT, D), jnp.float32),
    scratch_types=[
        pltpu.VMEM((NB, 2 * C, CIN, D), jnp.float32),
        pltpu.VMEM((NB, C, COUT, D), jnp.float32),
        pltpu.SemaphoreType.DMA,
        pltpu.SemaphoreType.DMA,
        pltpu.SemaphoreType.DMA,
        pltpu.SemaphoreType.DMA,
    ],
)(_body)


def kernel(x, y):
    return _tp(x, y)


# speedup vs baseline: 1.2086x; 1.0547x over previous
import functools

import jax
import jax.numpy as jnp
from jax import lax
from jax.experimental import pallas as pl
from jax.experimental.pallas import tpu as pltpu
from jax.experimental.pallas import tpu_sc as plsc

E, CIN, COUT, D = 8192, 4, 8, 512
L = 16
NC, NS = 2, 16
NW = NC * NS
EPW = E // NW
C = 4
NCH = EPW // C
NB = 2
JPE = D // L
INV_SQRT3 = 0.5773502691896258


def _body(x_hbm, y_hbm, o_hbm, xv, yv, ov,
          sx0, sx1, sy0, sy1, so0, so1):
    sx = (sx0, sx1)
    sy = (sy0, sy1)
    so = (so0, so1)
    wid = lax.axis_index("s") * NC + lax.axis_index("c")
    base = wid * EPW

    for b in range(NB):
        off = base + b * C
        pltpu.async_copy(x_hbm.at[pl.ds(off, C)], xv.at[b], sx[b])
        pltpu.async_copy(y_hbm.at[pl.ds(off, C)], yv.at[b], sy[b])

    def round_body(g, carry):
        for b in range(NB):
            ci = g * NB + b
            off = base + ci * C

            pltpu.make_async_copy(x_hbm.at[pl.ds(off, C)], xv.at[b], sx[b]).wait()
            pltpu.make_async_copy(y_hbm.at[pl.ds(off, C)], yv.at[b], sy[b]).wait()

            @pl.when(g > 0)
            def _():
                pltpu.make_async_copy(
                    ov.at[b], o_hbm.at[pl.ds(base, C)], so[b]).wait()

            def _edge(e, carry3):
                for j in range(JPE):
                    s = pl.ds(j * L, L)
                    x0 = xv[b, e, 0, s]
                    x1 = xv[b, e, 1, s]
                    x2 = xv[b, e, 2, s]
                    x3 = xv[b, e, 3, s]
                    y0 = yv[b, e, 0, s]
                    y1 = yv[b, e, 1, s]
                    y2 = yv[b, e, 2, s]
                    y3 = yv[b, e, 3, s]
                    ov[b, e, 0, s] = x0 * y0
                    ov[b, e, 1, s] = x0 * y1
                    ov[b, e, 2, s] = x0 * y2
                    ov[b, e, 3, s] = x0 * y3
                    ov[b, e, 4, s] = x1 * y0
                    ov[b, e, 5, s] = x2 * y0
                    ov[b, e, 6, s] = x3 * y0
                    ov[b, e, 7, s] = (x1 * y1 + x2 * y2 + x3 * y3) * INV_SQRT3
                return carry3

            lax.fori_loop(0, C, _edge, 0)

            pltpu.async_copy(ov.at[b], o_hbm.at[pl.ds(off, C)], so[b])

            @pl.when(ci + NB < NCH)
            def _():
                noff = off + NB * C
                pltpu.async_copy(x_hbm.at[pl.ds(noff, C)], xv.at[b], sx[b])
                pltpu.async_copy(y_hbm.at[pl.ds(noff, C)], yv.at[b], sy[b])

        return carry

    lax.fori_loop(0, NCH // NB, round_body, 0)

    for b in range(NB):
        pltpu.make_async_copy(ov.at[b], o_hbm.at[pl.ds(base, C)], so[b]).wait()


_tp = functools.partial(
    pl.kernel,
    mesh=plsc.VectorSubcoreMesh(core_axis_name="c", subcore_axis_name="s"),
    out_type=jax.ShapeDtypeStruct((E, COUT, D), jnp.float32),
    scratch_types=[
        pltpu.VMEM((NB, C, CIN, D), jnp.float32),
        pltpu.VMEM((NB, C, CIN, D), jnp.float32),
        pltpu.VMEM((NB, C, COUT, D), jnp.float32),
        pltpu.SemaphoreType.DMA,
        pltpu.SemaphoreType.DMA,
        pltpu.SemaphoreType.DMA,
        pltpu.SemaphoreType.DMA,
        pltpu.SemaphoreType.DMA,
        pltpu.SemaphoreType.DMA,
    ],
)(_body)


def kernel(x, y):
    return _tp(x, y)
